# baseline (speedup 1.0000x reference)
import functools

import jax
import jax.numpy as jnp
from jax import lax
from jax.experimental import pallas as pl
from jax.experimental.pallas import tpu as pltpu
from jax.experimental.pallas import tpu_sc as plsc

N_GRID_STATIC = 10000
N_MESH = 10000
D_IN = 128
D_HID = 256
D_OUT = 128
N_EDGES = 320000

NC = 2
NS = 16
NW = NC * NS
EB = 64
KB = 160
EPT = KB * EB
EPAD = NW * EPT
assert KB % 3 == 1, "3-deep SC ring below assumes KB = 3k + 1"
assert EPAD >= N_EDGES and EPT % 128 == 0
CNT_W = 16
ROWS_PER_SUB = 632
ACC_ROWS = NS * ROWS_PER_SUB
DUMMY_ROW = N_GRID_STATIC


def _sc_gather_scatter(mesh_features, pairs2, zrows, zcnt, ones_blk):
    mesh = plsc.VectorSubcoreMesh(core_axis_name="c", subcore_axis_name="s")

    @functools.partial(
        pl.kernel,
        out_type=(
            jax.ShapeDtypeStruct((NC, ACC_ROWS, D_IN), jnp.float32),
            jax.ShapeDtypeStruct((NC, ACC_ROWS, CNT_W), jnp.float32),
        ),
        mesh=mesh,
        compiler_params=pltpu.CompilerParams(use_tc_tiling_on_sc=False,
                                             needs_layout_passes=False),
        scratch_types=[
            pltpu.VMEM((EPT // 128, 128), jnp.int32),
            [pltpu.VMEM((EB, D_IN), jnp.float32) for _ in range(3)],
            [pltpu.VMEM((EB,), jnp.int32) for _ in range(3)],
            pltpu.VMEM((EB,), jnp.int32),
            pltpu.VMEM((EB, CNT_W), jnp.float32),
            pltpu.VMEM_SHARED((ACC_ROWS, D_IN), jnp.float32),
            pltpu.VMEM_SHARED((ACC_ROWS, CNT_W), jnp.float32),
            [pltpu.SemaphoreType.DMA for _ in range(3)],
        ],
    )
    def k(mesh_hbm, pairs_hbm, zrows_hbm, zcnt_hbm, ones_hbm,
          psum_hbm, pcnt_hbm,
          pairs_v, rows, s32, d32, ones_v, acc_sh, cnt_sh, sems):
        cid = lax.axis_index("c")
        sid = lax.axis_index("s")
        wid = sid * NC + cid
        base = sid * ROWS_PER_SUB

        pltpu.sync_copy(zrows_hbm, acc_sh.at[pl.ds(base, ROWS_PER_SUB)])
        pltpu.sync_copy(zcnt_hbm, cnt_sh.at[pl.ds(base, ROWS_PER_SUB)])
        pltpu.sync_copy(pairs_hbm.at[pl.ds(wid * (EPT // 128), EPT // 128)],
                        pairs_v)
        pltpu.sync_copy(ones_hbm, ones_v)
        plsc.subcore_barrier()

        def unpack(j, out_ref, *, hi):
            row = j // 2
            col = (j % 2) * EB
            for h in range(EB // 16):
                w = pairs_v[row, pl.ds(col + 16 * h, 16)]
                if hi:
                    out_ref[pl.ds(16 * h, 16)] = jax.lax.shift_right_logical(
                        w, jnp.int32(16))
                else:
                    out_ref[pl.ds(16 * h, 16)] = jnp.bitwise_and(
                        w, jnp.int32(0xFFFF))

        def issue(b, j):
            unpack(j, s32[b], hi=False)
            pltpu.async_copy(mesh_hbm.at[s32[b]], rows[b], sems[b])

        def drain(b, j):
            pltpu.make_async_copy(mesh_hbm.at[s32[b]], rows[b], sems[b]).wait()
            unpack(j, d32, hi=True)
            pltpu.sync_copy(rows[b], acc_sh.at[d32], add=True)
            pltpu.sync_copy(ones_v, cnt_sh.at[d32], add=True)

        issue(0, 0)
        issue(1, 1)

        @pl.loop(0, KB - 6, step=3)
        def _(j):
            issue(2, j + 2)
            drain(0, j)
            issue(0, j + 3)
            drain(1, j + 1)
            issue(1, j + 4)
            drain(2, j + 2)

        issue(2, KB - 2)
        drain(0, KB - 4)
        issue(0, KB - 1)
        drain(1, KB - 3)
        drain(2, KB - 2)
        drain(0, KB - 1)

        plsc.subcore_barrier()
        pltpu.sync_copy(acc_sh.at[pl.ds(base, ROWS_PER_SUB)],
                        psum_hbm.at[cid, pl.ds(base, ROWS_PER_SUB)])
        pltpu.sync_copy(cnt_sh.at[pl.ds(base, ROWS_PER_SUB)],
                        pcnt_hbm.at[cid, pl.ds(base, ROWS_PER_SUB)])

    return k(mesh_features, pairs2, zrows, zcnt, ones_blk)


def _mlp_body(p_ref, c_ref, w1_ref, b1_ref, w2_ref, b2_ref, o_ref):
    p = p_ref[0] + p_ref[1]
    cnt = c_ref[0, :, :1] + c_ref[1, :, :1]
    agg = p / jnp.maximum(cnt, 1.0)
    h = jnp.dot(agg.astype(jnp.bfloat16), w1_ref[...],
                preferred_element_type=jnp.float32)
    h = h + b1_ref[...]
    h = h * jax.nn.sigmoid(h)
    out = jnp.dot(h.astype(jnp.bfloat16), w2_ref[...],
                  preferred_element_type=jnp.float32)
    o_ref[...] = out + b2_ref[...]


def _tc_mean_mlp(psum, pcnt, w1t, b1, w2t, b2):
    R = 1000
    grid = (N_GRID_STATIC // R,)
    return pl.pallas_call(
        _mlp_body,
        grid=grid,
        in_specs=[
            pl.BlockSpec((NC, R, D_IN), lambda i: (0, i, 0)),
            pl.BlockSpec((NC, R, CNT_W), lambda i: (0, i, 0)),
            pl.BlockSpec((D_IN, D_HID), lambda i: (0, 0)),
            pl.BlockSpec((1, D_HID), lambda i: (0, 0)),
            pl.BlockSpec((D_HID, D_OUT), lambda i: (0, 0)),
            pl.BlockSpec((1, D_OUT), lambda i: (0, 0)),
        ],
        out_specs=pl.BlockSpec((R, D_OUT), lambda i: (i, 0)),
        out_shape=jax.ShapeDtypeStruct((N_GRID_STATIC, D_OUT), jnp.float32),
    )(psum, pcnt, w1t, b1.reshape(1, D_HID), w2t, b2.reshape(1, D_OUT))


def kernel(mesh_features, edge_index, n_grid_nodes, W1, b1, W2, b2):
    src = edge_index[0].astype(jnp.int32)
    off = jnp.asarray(n_grid_nodes).astype(jnp.int32) - jnp.int32(N_GRID_STATIC)
    dst = edge_index[1].astype(jnp.int32) + off

    pad = EPAD - N_EDGES
    pad_dst = DUMMY_ROW + jnp.arange(pad, dtype=jnp.int32) % (ACC_ROWS - DUMMY_ROW)
    pairs = jnp.concatenate([src, jnp.zeros((pad,), jnp.int32)]) | (
        jnp.concatenate([dst, pad_dst]) << 16)
    pairs2 = pairs.reshape(EPAD // 128, 128)

    zrows = jnp.zeros((ROWS_PER_SUB, D_IN), jnp.float32)
    zcnt = jnp.zeros((ROWS_PER_SUB, CNT_W), jnp.float32)
    ones_blk = jnp.ones((EB, CNT_W), jnp.float32)

    psum, pcnt = _sc_gather_scatter(mesh_features, pairs2, zrows, zcnt, ones_blk)
    return _tc_mean_mlp(psum, pcnt,
                        W1.T.astype(jnp.bfloat16), b1,
                        W2.T.astype(jnp.bfloat16), b2)


# speedup vs baseline: 2.1986x; 2.1986x over previous
import functools

import jax
import jax.numpy as jnp
from jax import lax
from jax.experimental import pallas as pl
from jax.experimental.pallas import tpu as pltpu
from jax.experimental.pallas import tpu_sc as plsc

N_GRID_STATIC = 10000
N_MESH = 10000
D_IN = 128
D_HID = 256
D_OUT = 128
N_EDGES = 320000

NC = 2
NS = 16
NW = NC * NS
EB = 64
KB = -(-N_EDGES // (NW * EB))
EPAD = NW * KB * EB
assert KB % 3 == 1, "3-deep SC ring below assumes KB = 3k + 1"
CNT_W = 16
ROWS_PER_SUB = 632
ACC_ROWS = NS * ROWS_PER_SUB
DUMMY_ROW = N_GRID_STATIC


def _widen_idx(idx16_ref, j, out_ref):
    for h in range(EB // 32):
        w = plsc.bitcast(idx16_ref[j, pl.ds(32 * h, 32)], jnp.int32)
        out_ref[pl.ds(32 * h, 16)] = jnp.bitwise_and(w, jnp.int32(0xFFFF))
        out_ref[pl.ds(32 * h + 16, 16)] = jax.lax.shift_right_logical(
            w, jnp.int32(16))


def _sc_gather_scatter(mesh_features, src3, dst3, zrows, zcnt, ones_blk):
    mesh = plsc.VectorSubcoreMesh(core_axis_name="c", subcore_axis_name="s")

    @functools.partial(
        pl.kernel,
        out_type=(
            jax.ShapeDtypeStruct((NC, ACC_ROWS, D_IN), jnp.float32),
            jax.ShapeDtypeStruct((NC, ACC_ROWS, CNT_W), jnp.float32),
        ),
        mesh=mesh,
        compiler_params=pltpu.CompilerParams(use_tc_tiling_on_sc=False,
                                             needs_layout_passes=False),
        scratch_types=[
            pltpu.VMEM((KB, EB), jnp.int16),
            pltpu.VMEM((KB, EB), jnp.int16),
            [pltpu.VMEM((EB, D_IN), jnp.float32) for _ in range(3)],
            [pltpu.VMEM((EB,), jnp.int32) for _ in range(3)],
            pltpu.VMEM((EB,), jnp.int32),
            pltpu.VMEM((EB, CNT_W), jnp.float32),
            pltpu.VMEM_SHARED((ACC_ROWS, D_IN), jnp.float32),
            pltpu.VMEM_SHARED((ACC_ROWS, CNT_W), jnp.float32),
            [pltpu.SemaphoreType.DMA for _ in range(3)],
        ],
    )
    def k(mesh_hbm, src_hbm, dst_hbm, zrows_hbm, zcnt_hbm, ones_hbm,
          psum_hbm, pcnt_hbm,
          src16_v, dst16_v, rows, s32, d32, ones_v, acc_sh, cnt_sh, sems):
        cid = lax.axis_index("c")
        sid = lax.axis_index("s")
        wid = sid * NC + cid
        base = sid * ROWS_PER_SUB

        pltpu.sync_copy(zrows_hbm, acc_sh.at[pl.ds(base, ROWS_PER_SUB)])
        pltpu.sync_copy(zcnt_hbm, cnt_sh.at[pl.ds(base, ROWS_PER_SUB)])
        pltpu.sync_copy(src_hbm.at[wid], src16_v)
        pltpu.sync_copy(dst_hbm.at[wid], dst16_v)
        pltpu.sync_copy(ones_hbm, ones_v)
        plsc.subcore_barrier()

        def issue(b, j):
            _widen_idx(src16_v, j, s32[b])
            pltpu.async_copy(mesh_hbm.at[s32[b]], rows[b], sems[b])

        def drain(b, j):
            pltpu.make_async_copy(mesh_hbm.at[s32[b]], rows[b], sems[b]).wait()
            _widen_idx(dst16_v, j, d32)
            pltpu.sync_copy(rows[b], acc_sh.at[d32], add=True)
            pltpu.sync_copy(ones_v, cnt_sh.at[d32], add=True)

        issue(0, 0)
        issue(1, 1)

        @pl.loop(0, KB - 6, step=3)
        def _(j):
            issue(2, j + 2)
            drain(0, j)
            issue(0, j + 3)
            drain(1, j + 1)
            issue(1, j + 4)
            drain(2, j + 2)

        issue(2, KB - 2)
        drain(0, KB - 4)
        issue(0, KB - 1)
        drain(1, KB - 3)
        drain(2, KB - 2)
        drain(0, KB - 1)

        plsc.subcore_barrier()
        pltpu.sync_copy(acc_sh.at[pl.ds(base, ROWS_PER_SUB)],
                        psum_hbm.at[cid, pl.ds(base, ROWS_PER_SUB)])
        pltpu.sync_copy(cnt_sh.at[pl.ds(base, ROWS_PER_SUB)],
                        pcnt_hbm.at[cid, pl.ds(base, ROWS_PER_SUB)])

    return k(mesh_features, src3, dst3, zrows, zcnt, ones_blk)


def _mlp_body(p_ref, c_ref, w1_ref, b1_ref, w2_ref, b2_ref, o_ref):
    p = p_ref[0] + p_ref[1]
    cnt = c_ref[0, :, :1] + c_ref[1, :, :1]
    agg = p / jnp.maximum(cnt, 1.0)
    h = jnp.dot(agg, w1_ref[...], preferred_element_type=jnp.float32)
    h = h + b1_ref[...]
    h = h * jax.nn.sigmoid(h)
    out = jnp.dot(h, w2_ref[...], preferred_element_type=jnp.float32)
    o_ref[...] = out + b2_ref[...]


def _tc_mean_mlp(psum, pcnt, w1t, b1, w2t, b2):
    R = 1000
    grid = (N_GRID_STATIC // R,)
    return pl.pallas_call(
        _mlp_body,
        grid=grid,
        in_specs=[
            pl.BlockSpec((NC, R, D_IN), lambda i: (0, i, 0)),
            pl.BlockSpec((NC, R, CNT_W), lambda i: (0, i, 0)),
            pl.BlockSpec((D_IN, D_HID), lambda i: (0, 0)),
            pl.BlockSpec((1, D_HID), lambda i: (0, 0)),
            pl.BlockSpec((D_HID, D_OUT), lambda i: (0, 0)),
            pl.BlockSpec((1, D_OUT), lambda i: (0, 0)),
        ],
        out_specs=pl.BlockSpec((R, D_OUT), lambda i: (i, 0)),
        out_shape=jax.ShapeDtypeStruct((N_GRID_STATIC, D_OUT), jnp.float32),
    )(psum, pcnt, w1t, b1.reshape(1, D_HID), w2t, b2.reshape(1, D_OUT))


def kernel(mesh_features, edge_index, n_grid_nodes, W1, b1, W2, b2):
    src = edge_index[0].astype(jnp.int32)
    off = jnp.asarray(n_grid_nodes).astype(jnp.int32) - jnp.int32(N_GRID_STATIC)
    dst = edge_index[1].astype(jnp.int32) + off

    pad = EPAD - N_EDGES
    pad_dst = DUMMY_ROW + jnp.arange(pad, dtype=jnp.int32) % (ACC_ROWS - DUMMY_ROW)
    src3 = jnp.concatenate(
        [src, jnp.zeros((pad,), jnp.int32)]).astype(jnp.int16).reshape(NW, KB, EB)
    dst3 = jnp.concatenate(
        [dst, pad_dst]).astype(jnp.int16).reshape(NW, KB, EB)

    zrows = jnp.zeros((ROWS_PER_SUB, D_IN), jnp.float32)
    zcnt = jnp.zeros((ROWS_PER_SUB, CNT_W), jnp.float32)
    ones_blk = jnp.ones((EB, CNT_W), jnp.float32)

    psum, pcnt = _sc_gather_scatter(mesh_features, src3, dst3,
                                    zrows, zcnt, ones_blk)
    return _tc_mean_mlp(psum, pcnt, W1.T, b1, W2.T, b2)
